# baseline (speedup 1.0000x reference)
import functools

import jax
import jax.numpy as jnp
from jax import lax
from jax.experimental import pallas as pl
from jax.experimental.pallas import tpu as pltpu
from jax.experimental.pallas import tpu_sc as plsc

D_MODEL = 1024
BATCH = 4096
_NC, _NS = 2, 16
_NW = _NC * _NS
_BPW = BATCH // _NW
_CH = 16
_NCH = _BPW // _CH
_NB = 7

_mesh = plsc.VectorSubcoreMesh(core_axis_name="c", subcore_axis_name="s")


@functools.partial(
    pl.kernel,
    mesh=_mesh,
    out_type=jax.ShapeDtypeStruct((BATCH, D_MODEL), jnp.float32),
    scratch_types=[
        pltpu.VMEM((_BPW,), jnp.int32),
        pltpu.VMEM((_NB, _CH, D_MODEL), jnp.float32),
    ]
    + [pltpu.SemaphoreType.DMA] * _NB
    + [pltpu.SemaphoreType.DMA] * _NB,
)
def _gather_kernel(t_hbm, pe_hbm, out_hbm, idx_v, rows_v, *sems):
    gsem = sems[:_NB]
    ssem = sems[_NB:]
    wid = lax.axis_index("s") * _NC + lax.axis_index("c")
    base = wid * _BPW
    pltpu.sync_copy(t_hbm.at[pl.ds(base, _BPW)], idx_v)

    sizes = [8, 8] + [16] * 6 + [8, 8]
    offs = [sum(sizes[:i]) for i in range(len(sizes))]
    ncH = len(sizes)

    def start_gather(c):
        b = c % _NB
        return pltpu.async_copy(
            pe_hbm.at[idx_v.at[pl.ds(offs[c], sizes[c])]],
            rows_v.at[b].at[pl.ds(0, sizes[c])], gsem[b])

    gathers = [None] * ncH
    scatters = [None] * ncH
    for c in range(min(_NB, ncH)):
        gathers[c] = start_gather(c)

    for c in range(ncH):
        b = c % _NB
        gathers[c].wait()
        scatters[c] = pltpu.async_copy(
            rows_v.at[b].at[pl.ds(0, sizes[c])],
            out_hbm.at[pl.ds(base + offs[c], sizes[c])], ssem[b])
        if c >= 1 and c - 1 + _NB < ncH:
            scatters[c - 1].wait()
            gathers[c - 1 + _NB] = start_gather(c - 1 + _NB)

    for c in range(ncH):
        if c + _NB >= ncH:
            scatters[c].wait()


def kernel(t, pe):
    return _gather_kernel(t, pe)


# speedup vs baseline: 1.0220x; 1.0220x over previous
import functools

import jax
import jax.numpy as jnp
from jax import lax
from jax.experimental import pallas as pl
from jax.experimental.pallas import tpu as pltpu
from jax.experimental.pallas import tpu_sc as plsc

D_MODEL = 1024
BATCH = 4096
_NC, _NS = 2, 16
_NW = _NC * _NS
_BPW = BATCH // _NW
_CH = 16
_NCH = _BPW // _CH
_NB = 7

_mesh = plsc.VectorSubcoreMesh(core_axis_name="c", subcore_axis_name="s")


@functools.partial(
    pl.kernel,
    mesh=_mesh,
    out_type=jax.ShapeDtypeStruct((BATCH, D_MODEL), jnp.float32),
    scratch_types=[
        pltpu.VMEM((_BPW,), jnp.int32),
        pltpu.VMEM((_NB, _CH, D_MODEL), jnp.float32),
    ]
    + [pltpu.SemaphoreType.DMA] * _NB
    + [pltpu.SemaphoreType.DMA] * _NB,
)
def _gather_kernel(t_hbm, pe_hbm, out_hbm, idx_v, rows_v, *sems):
    gsem = sems[:_NB]
    ssem = sems[_NB:]
    wid = lax.axis_index("s") * _NC + lax.axis_index("c")
    base = wid * _BPW
    pltpu.sync_copy(t_hbm.at[pl.ds(base, _BPW)], idx_v)

    def start_gather(c):
        b = c % _NB
        return pltpu.async_copy(
            pe_hbm.at[idx_v.at[pl.ds(c * _CH, _CH)]], rows_v.at[b], gsem[b])

    gathers = [None] * _NCH
    scatters = [None] * _NCH
    for c in range(min(_NB, _NCH)):
        gathers[c] = start_gather(c)

    for c in range(_NCH):
        if c >= 1 and c - 1 + _NB < _NCH:
            scatters[c - 1].wait()
            gathers[c - 1 + _NB] = start_gather(c - 1 + _NB)
        b = c % _NB
        gathers[c].wait()
        scatters[c] = pltpu.async_copy(
            rows_v.at[b], out_hbm.at[pl.ds(base + c * _CH, _CH)], ssem[b])

    for c in range(_NCH):
        if c + _NB >= _NCH:
            scatters[c].wait()


def kernel(t, pe):
    return _gather_kernel(t, pe)
